# baseline (speedup 1.0000x reference)
import jax
import jax.numpy as jnp
from jax import lax
from jax.experimental import pallas as pl
from jax.experimental.pallas import tpu as pltpu
from jax.experimental.pallas import tpu_sc as plsc

ROWS = 4096
COLS = 16384

R_SC = 1536

NUM_CORES = 2
NUM_SUBCORES = 16
NUM_WORKERS = NUM_CORES * NUM_SUBCORES

CHUNK = 16384
N_BUF = 4

TC_BM = 64


def _process_chunk(buf):
    lane = lax.iota(jnp.int32, 16)
    i0_init = lane * 4
    zero = jnp.zeros((16,), jnp.float32)

    @plsc.parallel_loop(0, CHUNK, step=64, unroll=4)
    def _loop(b):
        i0 = i0_init + b
        i1 = i0 + 1
        i2 = i0 + 2
        i3 = i0 + 3
        v0 = plsc.load_gather(buf, [i0])
        v1 = plsc.load_gather(buf, [i1])
        v2 = plsc.load_gather(buf, [i2])
        v3 = plsc.load_gather(buf, [i3])
        a0 = jnp.abs(v0)
        a1 = jnp.abs(v1)
        a2 = jnp.abs(v2)
        a3 = jnp.abs(v3)
        i01 = (a0 > a1).astype(jnp.int32)
        i02 = (a0 > a2).astype(jnp.int32)
        i03 = (a0 > a3).astype(jnp.int32)
        i12 = (a1 > a2).astype(jnp.int32)
        i13 = (a1 > a3).astype(jnp.int32)
        i23 = (a2 > a3).astype(jnp.int32)
        d0 = (i01 + i02) + i03 <= 1
        d1 = (i12 + i13) - i01 <= 0
        d2 = i23 - (i02 + i12) <= -1
        d3 = (i03 + i13) + i23 >= 2
        plsc.store_scatter(buf, [i0], zero, mask=d0)
        plsc.store_scatter(buf, [i1], zero, mask=d1)
        plsc.store_scatter(buf, [i2], zero, mask=d2)
        plsc.store_scatter(buf, [i3], zero, mask=d3)


def _make_sc_call(rows):
    total = rows * COLS
    per_worker = total // NUM_WORKERS
    n_chunks = per_worker // CHUNK

    def body(x_hbm, o_hbm, *refs):
        cid = lax.axis_index("c")
        sid = lax.axis_index("s")
        wid = sid * NUM_CORES + cid
        base = wid * per_worker

        bufs = refs[:N_BUF]
        in_sems = refs[N_BUF:2 * N_BUF]
        out_sems = refs[2 * N_BUF:3 * N_BUF]

        def src(j):
            return x_hbm.at[pl.ds(base + j * CHUNK, CHUNK)]

        def dst(j):
            return o_hbm.at[pl.ds(base + j * CHUNK, CHUNK)]

        def do_chunk(j, par, wait_out_prev, issue_next):
            p = par % N_BUF
            pltpu.make_async_copy(src(j), bufs[p], in_sems[p]).wait()
            _process_chunk(bufs[p])
            pltpu.async_copy(bufs[p], dst(j), out_sems[p])
            if wait_out_prev:
                q = (par - 1) % N_BUF
                pltpu.make_async_copy(bufs[q], dst(j - 1), out_sems[q]).wait()
            if issue_next:
                n = (par + 3) % N_BUF
                pltpu.async_copy(src(j + 3), bufs[n], in_sems[n])

        for p in range(3):
            pltpu.async_copy(src(p), bufs[p], in_sems[p])
        do_chunk(0, 0, False, True)
        for k in range(1, N_BUF):
            do_chunk(k, k, True, True)

        def round_body(r, _):
            j0 = r * N_BUF
            for k in range(N_BUF):
                do_chunk(j0 + k, k, True, True)
            return 0

        lax.fori_loop(1, n_chunks // N_BUF - 1, round_body, 0)

        j0 = n_chunks - N_BUF
        do_chunk(j0, 0, True, True)
        for k in range(1, N_BUF):
            do_chunk(j0 + k, k, True, False)
        q = (n_chunks - 1) % N_BUF
        pltpu.make_async_copy(bufs[q], dst(n_chunks - 1), out_sems[q]).wait()

    mesh = plsc.VectorSubcoreMesh(
        core_axis_name="c", subcore_axis_name="s",
        num_cores=NUM_CORES, num_subcores=NUM_SUBCORES,
    )
    return pl.kernel(
        body,
        out_type=jax.ShapeDtypeStruct((total,), jnp.float32),
        mesh=mesh,
        scratch_types=(
            [pltpu.VMEM((CHUNK,), jnp.float32)] * N_BUF
            + [pltpu.SemaphoreType.DMA] * (2 * N_BUF)
        ),
        compiler_params=pltpu.CompilerParams(needs_layout_passes=False),
    )


def _tc_body(x_ref, o_ref):
    x = x_ref[...]
    a = jnp.abs(x)
    p = lax.broadcasted_iota(jnp.int32, x.shape, 1) & 3
    rank = jnp.zeros(x.shape, jnp.int32)
    for d in (1, 2, 3):
        fwd = pltpu.roll(a, COLS - d, 1)
        bwd = pltpu.roll(a, d, 1)
        rank = rank + jnp.where((p <= 3 - d) & (a > fwd), 1, 0)
        rank = rank + jnp.where((p >= d) & (a >= bwd), 1, 0)
    o_ref[...] = jnp.where(rank >= 2, x, 0.0)


def _tc_call(x):
    r = x.shape[0]
    return pl.pallas_call(
        _tc_body,
        out_shape=jax.ShapeDtypeStruct((r, COLS), jnp.float32),
        grid=(r // TC_BM,),
        in_specs=[pl.BlockSpec((TC_BM, COLS), lambda i: (i, 0))],
        out_specs=pl.BlockSpec((TC_BM, COLS), lambda i: (i, 0)),
    )(x)


@jax.jit
def kernel(inputs):
    parts = []
    if R_SC > 0:
        sc_out = _make_sc_call(R_SC)(inputs[:R_SC].reshape(R_SC * COLS))
        parts.append(sc_out.reshape(R_SC, COLS))
    if R_SC < ROWS:
        parts.append(_tc_call(inputs[R_SC:]))
    if len(parts) == 1:
        return parts[0]
    return jnp.concatenate(parts, axis=0)


# speedup vs baseline: 1.2109x; 1.2109x over previous
import jax
import jax.numpy as jnp
from jax import lax
from jax.experimental import pallas as pl
from jax.experimental.pallas import tpu as pltpu
from jax.experimental.pallas import tpu_sc as plsc

ROWS = 4096
COLS = 16384

R_SC = 2432

NUM_CORES = 2
NUM_SUBCORES = 16
NUM_WORKERS = NUM_CORES * NUM_SUBCORES

CHUNK = 16384
N_BUF = 4

TC_BM = 64


def _process_chunk(buf):
    lane = lax.iota(jnp.int32, 16)
    i0_init = lane * 4
    zero = jnp.zeros((16,), jnp.float32)

    @plsc.parallel_loop(0, CHUNK, step=64, unroll=4)
    def _loop(b):
        i0 = i0_init + b
        i1 = i0 + 1
        i2 = i0 + 2
        i3 = i0 + 3
        v0 = plsc.load_gather(buf, [i0])
        v1 = plsc.load_gather(buf, [i1])
        v2 = plsc.load_gather(buf, [i2])
        v3 = plsc.load_gather(buf, [i3])
        a0 = jnp.abs(v0)
        a1 = jnp.abs(v1)
        a2 = jnp.abs(v2)
        a3 = jnp.abs(v3)
        i01 = (a0 > a1).astype(jnp.int32)
        i02 = (a0 > a2).astype(jnp.int32)
        i03 = (a0 > a3).astype(jnp.int32)
        i12 = (a1 > a2).astype(jnp.int32)
        i13 = (a1 > a3).astype(jnp.int32)
        i23 = (a2 > a3).astype(jnp.int32)
        d0 = (i01 + i02) + i03 <= 1
        d1 = (i12 + i13) - i01 <= 0
        d2 = i23 - (i02 + i12) <= -1
        d3 = (i03 + i13) + i23 >= 2
        plsc.store_scatter(buf, [i0], zero, mask=d0)
        plsc.store_scatter(buf, [i1], zero, mask=d1)
        plsc.store_scatter(buf, [i2], zero, mask=d2)
        plsc.store_scatter(buf, [i3], zero, mask=d3)


def _make_sc_call(rows):
    total = ROWS * COLS
    per_worker = (rows * COLS) // NUM_WORKERS
    n_chunks = per_worker // CHUNK

    def body(x_hbm, o_hbm, *refs):
        cid = lax.axis_index("c")
        sid = lax.axis_index("s")
        wid = sid * NUM_CORES + cid
        base = wid * per_worker

        bufs = refs[:N_BUF]
        in_sems = refs[N_BUF:2 * N_BUF]
        out_sems = refs[2 * N_BUF:3 * N_BUF]

        def src(j):
            return x_hbm.at[pl.ds(base + j * CHUNK, CHUNK)]

        def dst(j):
            return o_hbm.at[pl.ds(base + j * CHUNK, CHUNK)]

        def do_chunk(j, par, wait_out_prev, issue_next):
            p = par % N_BUF
            pltpu.make_async_copy(src(j), bufs[p], in_sems[p]).wait()
            _process_chunk(bufs[p])
            pltpu.async_copy(bufs[p], dst(j), out_sems[p])
            if wait_out_prev:
                q = (par - 1) % N_BUF
                pltpu.make_async_copy(bufs[q], dst(j - 1), out_sems[q]).wait()
            if issue_next:
                n = (par + 3) % N_BUF
                pltpu.async_copy(src(j + 3), bufs[n], in_sems[n])

        for p in range(3):
            pltpu.async_copy(src(p), bufs[p], in_sems[p])
        do_chunk(0, 0, False, True)
        for k in range(1, N_BUF):
            do_chunk(k, k, True, True)

        def round_body(r, _):
            j0 = r * N_BUF
            for k in range(N_BUF):
                do_chunk(j0 + k, k, True, True)
            return 0

        lax.fori_loop(1, n_chunks // N_BUF - 1, round_body, 0)

        j0 = n_chunks - N_BUF
        do_chunk(j0, 0, True, True)
        for k in range(1, N_BUF):
            do_chunk(j0 + k, k, True, False)
        q = (n_chunks - 1) % N_BUF
        pltpu.make_async_copy(bufs[q], dst(n_chunks - 1), out_sems[q]).wait()

    mesh = plsc.VectorSubcoreMesh(
        core_axis_name="c", subcore_axis_name="s",
        num_cores=NUM_CORES, num_subcores=NUM_SUBCORES,
    )
    return pl.kernel(
        body,
        out_type=jax.ShapeDtypeStruct((total,), jnp.float32),
        mesh=mesh,
        scratch_types=(
            [pltpu.VMEM((CHUNK,), jnp.float32)] * N_BUF
            + [pltpu.SemaphoreType.DMA] * (2 * N_BUF)
        ),
        compiler_params=pltpu.CompilerParams(needs_layout_passes=False),
    )


def _tc_body(x_ref, o_ref):
    x = x_ref[...]
    a = jnp.abs(x)
    p = lax.broadcasted_iota(jnp.int32, x.shape, 1) & 3
    rank = jnp.zeros(x.shape, jnp.int32)
    for d in (1, 2, 3):
        fwd = pltpu.roll(a, COLS - d, 1)
        bwd = pltpu.roll(a, d, 1)
        rank = rank + jnp.where((p <= 3 - d) & (a > fwd), 1, 0)
        rank = rank + jnp.where((p >= d) & (a >= bwd), 1, 0)
    o_ref[...] = jnp.where(rank >= 2, x, 0.0)


def _tc_call(x, row0):
    r = ROWS - row0
    b0 = row0 // TC_BM
    return pl.pallas_call(
        _tc_body,
        out_shape=jax.ShapeDtypeStruct((r, COLS), jnp.float32),
        grid=(r // TC_BM,),
        in_specs=[pl.BlockSpec((TC_BM, COLS), lambda i: (i + b0, 0))],
        out_specs=pl.BlockSpec((TC_BM, COLS), lambda i: (i, 0)),
    )(x)


@jax.jit
def kernel(inputs):
    if R_SC == ROWS:
        flat = _make_sc_call(ROWS)(inputs.reshape(ROWS * COLS))
        return flat.reshape(ROWS, COLS)
    if R_SC == 0:
        return _tc_call(inputs, 0)
    sc_full = _make_sc_call(R_SC)(inputs.reshape(ROWS * COLS))
    tc_out = _tc_call(inputs, R_SC)
    return lax.dynamic_update_slice(
        sc_full.reshape(ROWS, COLS), tc_out, (R_SC, 0))


# speedup vs baseline: 4.1485x; 3.4259x over previous
import jax
import jax.numpy as jnp
from jax import lax
from jax.experimental import pallas as pl
from jax.experimental.pallas import tpu as pltpu
from jax.experimental.pallas import tpu_sc as plsc

ROWS = 4096
COLS = 16384

R_SC = 4096

NUM_CORES = 2
NUM_SUBCORES = 16
NUM_WORKERS = NUM_CORES * NUM_SUBCORES

CHUNK = 16384
N_BUF = 4

TC_BM = 64


def _process_chunk(buf):
    lane = lax.iota(jnp.int32, 16)
    i0_init = lane * 4
    zero = jnp.zeros((16,), jnp.float32)

    @plsc.parallel_loop(0, CHUNK, step=64, unroll=4)
    def _loop(b):
        i0 = i0_init + b
        i1 = i0 + 1
        i2 = i0 + 2
        i3 = i0 + 3
        v0 = plsc.load_gather(buf, [i0])
        v1 = plsc.load_gather(buf, [i1])
        v2 = plsc.load_gather(buf, [i2])
        v3 = plsc.load_gather(buf, [i3])
        a0 = jnp.abs(v0)
        a1 = jnp.abs(v1)
        a2 = jnp.abs(v2)
        a3 = jnp.abs(v3)
        i01 = (a0 > a1).astype(jnp.int32)
        i02 = (a0 > a2).astype(jnp.int32)
        i03 = (a0 > a3).astype(jnp.int32)
        i12 = (a1 > a2).astype(jnp.int32)
        i13 = (a1 > a3).astype(jnp.int32)
        i23 = (a2 > a3).astype(jnp.int32)
        d0 = (i01 + i02) + i03 <= 1
        d1 = (i12 + i13) - i01 <= 0
        d2 = i23 - (i02 + i12) <= -1
        d3 = (i03 + i13) + i23 >= 2
        plsc.store_scatter(buf, [i0], zero, mask=d0)
        plsc.store_scatter(buf, [i1], zero, mask=d1)
        plsc.store_scatter(buf, [i2], zero, mask=d2)
        plsc.store_scatter(buf, [i3], zero, mask=d3)


def _make_sc_call(rows):
    n_chunks = rows // NUM_WORKERS

    def body(x_hbm, o_hbm, *refs):
        cid = lax.axis_index("c")
        sid = lax.axis_index("s")
        wid = sid * NUM_CORES + cid
        base = wid * n_chunks

        bufs = refs[:N_BUF]
        in_sems = refs[N_BUF:2 * N_BUF]
        out_sems = refs[2 * N_BUF:3 * N_BUF]

        def src(j):
            return x_hbm.at[base + j]

        def dst(j):
            return o_hbm.at[base + j]

        def do_chunk(j, par, wait_out_prev, issue_next):
            p = par % N_BUF
            pltpu.make_async_copy(src(j), bufs[p], in_sems[p]).wait()
            _process_chunk(bufs[p])
            pltpu.async_copy(bufs[p], dst(j), out_sems[p])
            if wait_out_prev:
                q = (par - 1) % N_BUF
                pltpu.make_async_copy(bufs[q], dst(j - 1), out_sems[q]).wait()
            if issue_next:
                n = (par + 3) % N_BUF
                pltpu.async_copy(src(j + 3), bufs[n], in_sems[n])

        for p in range(3):
            pltpu.async_copy(src(p), bufs[p], in_sems[p])
        do_chunk(0, 0, False, True)
        for k in range(1, N_BUF):
            do_chunk(k, k, True, True)

        def round_body(r, _):
            j0 = r * N_BUF
            for k in range(N_BUF):
                do_chunk(j0 + k, k, True, True)
            return 0

        lax.fori_loop(1, n_chunks // N_BUF - 1, round_body, 0)

        j0 = n_chunks - N_BUF
        do_chunk(j0, 0, True, True)
        for k in range(1, N_BUF):
            do_chunk(j0 + k, k, True, False)
        q = (n_chunks - 1) % N_BUF
        pltpu.make_async_copy(bufs[q], dst(n_chunks - 1), out_sems[q]).wait()

    mesh = plsc.VectorSubcoreMesh(
        core_axis_name="c", subcore_axis_name="s",
        num_cores=NUM_CORES, num_subcores=NUM_SUBCORES,
    )
    return pl.kernel(
        body,
        out_type=jax.ShapeDtypeStruct((ROWS, COLS), jnp.float32),
        mesh=mesh,
        scratch_types=(
            [pltpu.VMEM((CHUNK,), jnp.float32)] * N_BUF
            + [pltpu.SemaphoreType.DMA] * (2 * N_BUF)
        ),
        compiler_params=pltpu.CompilerParams(needs_layout_passes=False),
    )


def _tc_body(x_ref, o_ref):
    x = x_ref[...]
    a = jnp.abs(x)
    p = lax.broadcasted_iota(jnp.int32, x.shape, 1) & 3
    rank = jnp.zeros(x.shape, jnp.int32)
    for d in (1, 2, 3):
        fwd = pltpu.roll(a, COLS - d, 1)
        bwd = pltpu.roll(a, d, 1)
        rank = rank + jnp.where((p <= 3 - d) & (a > fwd), 1, 0)
        rank = rank + jnp.where((p >= d) & (a >= bwd), 1, 0)
    o_ref[...] = jnp.where(rank >= 2, x, 0.0)


def _tc_call(x, row0):
    r = ROWS - row0
    b0 = row0 // TC_BM
    return pl.pallas_call(
        _tc_body,
        out_shape=jax.ShapeDtypeStruct((r, COLS), jnp.float32),
        grid=(r // TC_BM,),
        in_specs=[pl.BlockSpec((TC_BM, COLS), lambda i: (i + b0, 0))],
        out_specs=pl.BlockSpec((TC_BM, COLS), lambda i: (i, 0)),
    )(x)


@jax.jit
def kernel(inputs):
    if R_SC == ROWS:
        return _make_sc_call(ROWS)(inputs)
    if R_SC == 0:
        return _tc_call(inputs, 0)
    sc_full = _make_sc_call(R_SC)(inputs)
    tc_out = _tc_call(inputs, R_SC)
    return lax.dynamic_update_slice(sc_full, tc_out, (R_SC, 0))
